# baseline (speedup 1.0000x reference)
import functools

import jax
import jax.numpy as jnp
from jax import lax
from jax.experimental import pallas as pl
from jax.experimental.pallas import tpu as pltpu
from jax.experimental.pallas import tpu_sc as plsc

N = 8192
E = 8
D = 512
DH = D // 2
ROW = E + E * D

TB = 128
GRID = N // TB
SC_BLOCKS = 16
TC_BLOCKS = GRID - SC_BLOCKS
NT = TC_BLOCKS * TB
NSC = SC_BLOCKS * TB

NC = 2
NS = 16
L = 16

SEL_BLK = 512
SEL_GRID = NSC // SEL_BLK


def _fused_body(pT_ref, uT_ref, eps_ref, out_ref):
    u = uT_ref[...]
    uc = jnp.clip(u, 1e-6, 1.0 - 1e-6)
    g = -jnp.log(-jnp.log(uc))
    s = pT_ref[0:E, :] + g
    m = jnp.max(s, axis=0, keepdims=True)
    idx = lax.broadcasted_iota(jnp.int32, (E, TB), 0)
    sel = jnp.min(jnp.where(s == m, idx, E), axis=0, keepdims=True)

    mean = pT_ref[E:E + DH, :]
    lsd = pT_ref[E + DH:E + D, :]
    for e in range(1, E):
        msk = sel == e
        mean = jnp.where(msk, pT_ref[E + e * D:E + e * D + DH, :], mean)
        lsd = jnp.where(msk, pT_ref[E + e * D + DH:E + (e + 1) * D, :], lsd)
    mean_t = jnp.transpose(mean, (1, 0))
    lsd_t = jnp.transpose(lsd, (1, 0))
    out_ref[...] = mean_t + eps_ref[...] * jnp.exp(lsd_t)


def _selector_cols_body(lg_ref, u_ref, col_ref):
    u = u_ref[...]
    uc = jnp.clip(u, 1e-6, 1.0 - 1e-6)
    g = -jnp.log(-jnp.log(uc))
    s = lg_ref[...] + g
    m = jnp.max(s, axis=0, keepdims=True)
    idx = lax.broadcasted_iota(jnp.int32, (E, SEL_BLK), 0)
    sel = jnp.min(jnp.where(s == m, idx, E), axis=0)
    col_ref[...] = E + sel * D


def _sc_select_body(pT_hbm, col_hbm, eps_hbm, out_hbm,
                    col_v, mslab, lslab, eps_v, out_v,
                    sem_m0, sem_m1, sem_l0, sem_l1, sem_e):
    sem_m = (sem_m0, sem_m1)
    sem_l = (sem_l0, sem_l1)
    c = lax.axis_index("c")
    s = lax.axis_index("s")
    w = s * NC + c
    b = w // 2
    h = w % 2
    tok0 = b * TB
    gtok = NT + tok0
    co = h * 128

    pltpu.sync_copy(col_hbm.at[pl.ds(pl.multiple_of(tok0, 8), TB)], col_v)
    ec = pltpu.async_copy(
        eps_hbm.at[pl.ds(pl.multiple_of(gtok, 8), TB),
                   pl.ds(pl.multiple_of(co, 128), 128)], eps_v, sem_e)

    def slabs(e):
        rm = pl.multiple_of(E + e * D + co, 8)
        lanes = pl.ds(pl.multiple_of(gtok, 128), TB)
        return (pT_hbm.at[pl.ds(rm, 128), lanes],
                pT_hbm.at[pl.ds(rm + DH, 128), lanes])

    def issue(e, k):
        srm, srl = slabs(e)
        pltpu.async_copy(srm, mslab.at[k], sem_m[k])
        pltpu.async_copy(srl, lslab.at[k], sem_l[k])

    issue(0, 0)
    ec.wait()

    def epair(ep, carry):
        for k in (0, 1):
            e = 2 * ep + k
            en = e + 1

            @pl.when(en < E)
            def _():
                issue(en, 1 - k)

            srm, srl = slabs(e)
            pltpu.make_async_copy(srm, mslab.at[k], sem_m[k]).wait()
            pltpu.make_async_copy(srl, lslab.at[k], sem_l[k]).wait()
            cbase = E + e * D
            iota = lax.iota(jnp.int32, L)
            kvec = jnp.full((L,), k, jnp.int32)

            def gloop(g, carry2):
                colv = col_v[pl.ds(pl.multiple_of(g * L, L), L)]
                for j in range(L):
                    t = g * L + j

                    @pl.when(colv[j] == cbase)
                    def _():
                        tvec = jnp.full((L,), 0, jnp.int32) + t
                        for v in range(128 // L):
                            ds16 = pl.ds(v * L, L)
                            rvec = v * L + iota
                            mvec = plsc.load_gather(mslab, [kvec, rvec, tvec])
                            lvec = plsc.load_gather(lslab, [kvec, rvec, tvec])
                            out_v[t, ds16] = (
                                mvec + eps_v[t, ds16] * jnp.exp(lvec))
                return carry2

            lax.fori_loop(0, TB // L, gloop, 0)
        return carry

    lax.fori_loop(0, E // 2, epair, 0)
    pltpu.sync_copy(out_v,
                    out_hbm.at[pl.ds(pl.multiple_of(tok0, 8), TB),
                               pl.ds(pl.multiple_of(co, 128), 128)])


@functools.lru_cache(maxsize=1)
def _build_sc_select():
    return pl.kernel(
        _sc_select_body,
        out_type=jax.ShapeDtypeStruct((NSC, DH), jnp.float32),
        mesh=plsc.VectorSubcoreMesh(
            core_axis_name="c", subcore_axis_name="s",
            num_cores=NC, num_subcores=NS),
        compiler_params=pltpu.CompilerParams(
            use_tc_tiling_on_sc=True, needs_layout_passes=False),
        scratch_types=[
            pltpu.VMEM((TB,), jnp.int32),
            pltpu.VMEM((2, TB, 128), jnp.float32),
            pltpu.VMEM((2, TB, 128), jnp.float32),
            pltpu.VMEM((TB, 128), jnp.float32),
            pltpu.VMEM((TB, 128), jnp.float32),
            pltpu.SemaphoreType.DMA,
            pltpu.SemaphoreType.DMA,
            pltpu.SemaphoreType.DMA,
            pltpu.SemaphoreType.DMA,
            pltpu.SemaphoreType.DMA,
        ],
    )


def kernel(params, u, eps):
    pT = params.T
    uT = u.T
    cols_sc = pl.pallas_call(
        _selector_cols_body,
        grid=(SEL_GRID,),
        in_specs=[
            pl.BlockSpec((E, SEL_BLK), lambda b: (0, TC_BLOCKS * TB // SEL_BLK + b)),
            pl.BlockSpec((E, SEL_BLK), lambda b: (0, TC_BLOCKS * TB // SEL_BLK + b)),
        ],
        out_specs=pl.BlockSpec((SEL_BLK,), lambda b: (b,)),
        out_shape=jax.ShapeDtypeStruct((NSC,), jnp.int32),
    )(pT, uT)
    out_sc = _build_sc_select()(pT, cols_sc, eps)
    out_tc = pl.pallas_call(
        _fused_body,
        grid=(TC_BLOCKS,),
        in_specs=[
            pl.BlockSpec((ROW, TB), lambda b: (0, b)),
            pl.BlockSpec((E, TB), lambda b: (0, b)),
            pl.BlockSpec((TB, DH), lambda b: (b, 0)),
        ],
        out_specs=pl.BlockSpec((TB, DH), lambda b: (b, 0)),
        out_shape=jax.ShapeDtypeStruct((NT, DH), jnp.float32),
    )(pT, uT, eps)
    return jnp.concatenate([out_tc, out_sc], axis=0)


# speedup vs baseline: 1.9534x; 1.9534x over previous
import jax
import jax.numpy as jnp
from jax import lax
from jax.experimental import pallas as pl

N = 8192
E = 8
D = 512
DH = D // 2
ROW = E + E * D

TB = 256
GRID = N // TB


def _fused_body(pT_ref, uT_ref, eps_ref, out_ref):
    u = uT_ref[...]
    uc = jnp.clip(u, 1e-6, 1.0 - 1e-6)
    g = -jnp.log(-jnp.log(uc))
    s = pT_ref[0:E, :] + g
    m = jnp.max(s, axis=0, keepdims=True)
    idx = lax.broadcasted_iota(jnp.int32, (E, TB), 0)
    sel = jnp.min(jnp.where(s == m, idx, E), axis=0, keepdims=True)

    mean = pT_ref[E:E + DH, :]
    lsd = pT_ref[E + DH:E + D, :]
    for e in range(1, E):
        msk = sel == e
        mean = jnp.where(msk, pT_ref[E + e * D:E + e * D + DH, :], mean)
        lsd = jnp.where(msk, pT_ref[E + e * D + DH:E + (e + 1) * D, :], lsd)
    mean_t = jnp.transpose(mean, (1, 0))
    lsd_t = jnp.transpose(lsd, (1, 0))
    out_ref[...] = mean_t + eps_ref[...] * jnp.exp(lsd_t)


def kernel(params, u, eps):
    pT = params.T
    uT = u.T
    return pl.pallas_call(
        _fused_body,
        grid=(GRID,),
        in_specs=[
            pl.BlockSpec((ROW, TB), lambda b: (0, b)),
            pl.BlockSpec((E, TB), lambda b: (0, b)),
            pl.BlockSpec((TB, DH), lambda b: (b, 0)),
        ],
        out_specs=pl.BlockSpec((TB, DH), lambda b: (b, 0)),
        out_shape=jax.ShapeDtypeStruct((N, DH), jnp.float32),
    )(pT, uT, eps)


# speedup vs baseline: 2.0099x; 1.0289x over previous
import jax
import jax.numpy as jnp
from jax import lax
from jax.experimental import pallas as pl

N = 8192
E = 8
D = 512
DH = D // 2
ROW = E + E * D

TB = 512
GRID = N // TB


def _fused_body(pT_ref, uT_ref, eps_ref, out_ref):
    u = uT_ref[...]
    uc = jnp.clip(u, 1e-6, 1.0 - 1e-6)
    g = -jnp.log(-jnp.log(uc))
    s = pT_ref[0:E, :] + g
    m = jnp.max(s, axis=0, keepdims=True)
    idx = lax.broadcasted_iota(jnp.int32, (E, TB), 0)
    sel = jnp.min(jnp.where(s == m, idx, E), axis=0, keepdims=True)

    mean = pT_ref[E:E + DH, :]
    lsd = pT_ref[E + DH:E + D, :]
    for e in range(1, E):
        msk = sel == e
        mean = jnp.where(msk, pT_ref[E + e * D:E + e * D + DH, :], mean)
        lsd = jnp.where(msk, pT_ref[E + e * D + DH:E + (e + 1) * D, :], lsd)
    mean_t = jnp.transpose(mean, (1, 0))
    lsd_t = jnp.transpose(lsd, (1, 0))
    out_ref[...] = mean_t + eps_ref[...] * jnp.exp(lsd_t)


def kernel(params, u, eps):
    pT = params.T
    uT = u.T
    return pl.pallas_call(
        _fused_body,
        grid=(GRID,),
        in_specs=[
            pl.BlockSpec((ROW, TB), lambda b: (0, b)),
            pl.BlockSpec((E, TB), lambda b: (0, b)),
            pl.BlockSpec((TB, DH), lambda b: (b, 0)),
        ],
        out_specs=pl.BlockSpec((TB, DH), lambda b: (b, 0)),
        out_shape=jax.ShapeDtypeStruct((N, DH), jnp.float32),
    )(pT, uT, eps)
